# Initial kernel scaffold; baseline (speedup 1.0000x reference)
import jax
import jax.numpy as jnp
from jax.experimental import pallas as pl


def kernel(inputs, W, b):
    raise NotImplementedError("write your pallas kernel here")


# speedup vs baseline: 1.4420x; 1.4420x over previous
import jax
import jax.numpy as jnp
from jax.experimental import pallas as pl
from jax.experimental.pallas import tpu as pltpu

N = 16384
D = 2048
E = 64
BN = 512


def _router_kernel(x_ref, wt_ref, b_ref, idx_ref, gate_ref, prob_ref):
    x = x_ref[...]
    wt = wt_ref[...]
    b = b_ref[...]
    logits = jnp.dot(x, wt, preferred_element_type=jnp.float32) + b

    m = jnp.max(logits, axis=-1, keepdims=True)
    e = jnp.exp(logits - m)
    prob_ref[...] = e / jnp.sum(e, axis=-1, keepdims=True)

    i1 = jnp.argmax(logits, axis=-1)
    v1 = jnp.max(logits, axis=-1)
    lane = jax.lax.broadcasted_iota(jnp.int32, logits.shape, 1)
    masked = jnp.where(lane == i1[:, None], -jnp.inf, logits)
    i2 = jnp.argmax(masked, axis=-1)
    v2 = jnp.max(masked, axis=-1)

    idx_ref[...] = jnp.stack([i1, i2], axis=-1).astype(jnp.int32)

    g2 = 1.0 / (1.0 + jnp.exp(v1 - v2))
    g1 = 1.0 - g2
    gate_ref[...] = jnp.stack([g1, g2], axis=-1)


def kernel(inputs, W, b):
    wt = W.T
    b2 = b.reshape(1, E)
    grid = (N // BN,)
    out = pl.pallas_call(
        _router_kernel,
        grid=grid,
        in_specs=[
            pl.BlockSpec((BN, D), lambda i: (i, 0)),
            pl.BlockSpec((D, E), lambda i: (0, 0)),
            pl.BlockSpec((1, E), lambda i: (0, 0)),
        ],
        out_specs=[
            pl.BlockSpec((BN, 2), lambda i: (i, 0)),
            pl.BlockSpec((BN, 2), lambda i: (i, 0)),
            pl.BlockSpec((BN, E), lambda i: (i, 0)),
        ],
        out_shape=[
            jax.ShapeDtypeStruct((N, 2), jnp.int32),
            jax.ShapeDtypeStruct((N, 2), jnp.float32),
            jax.ShapeDtypeStruct((N, E), jnp.float32),
        ],
        compiler_params=pltpu.CompilerParams(
            dimension_semantics=("arbitrary",),
        ),
    )(inputs, wt, b2)
    topk_indices, topk_gates, all_probabilities = out
    return (topk_indices, topk_gates, all_probabilities)


# speedup vs baseline: 1.6171x; 1.1214x over previous
import jax
import jax.numpy as jnp
from jax.experimental import pallas as pl
from jax.experimental.pallas import tpu as pltpu

N = 16384
D = 2048
E = 64
BN = 1024


def _router_kernel(x_ref, wt_ref, b_ref, idx_ref, gate_ref, prob_ref):
    x = x_ref[...]
    wt = wt_ref[...]
    b = b_ref[...]
    logits = jnp.dot(x, wt, preferred_element_type=jnp.float32) + b

    m = jnp.max(logits, axis=-1, keepdims=True)
    e = jnp.exp(logits - m)
    prob_ref[...] = e / jnp.sum(e, axis=-1, keepdims=True)

    i1 = jnp.argmax(logits, axis=-1)
    v1 = jnp.max(logits, axis=-1)
    lane = jax.lax.broadcasted_iota(jnp.int32, logits.shape, 1)
    masked = jnp.where(lane == i1[:, None], -jnp.inf, logits)
    i2 = jnp.argmax(masked, axis=-1)
    v2 = jnp.max(masked, axis=-1)

    idx_ref[...] = jnp.stack([i1, i2], axis=-1).astype(jnp.int32)

    g2 = 1.0 / (1.0 + jnp.exp(v1 - v2))
    g1 = 1.0 - g2
    gate_ref[...] = jnp.stack([g1, g2], axis=-1)


def kernel(inputs, W, b):
    wt = W.T
    b2 = b.reshape(1, E)
    grid = (N // BN,)
    out = pl.pallas_call(
        _router_kernel,
        grid=grid,
        in_specs=[
            pl.BlockSpec((BN, D), lambda i: (i, 0)),
            pl.BlockSpec((D, E), lambda i: (0, 0)),
            pl.BlockSpec((1, E), lambda i: (0, 0)),
        ],
        out_specs=[
            pl.BlockSpec((BN, 2), lambda i: (i, 0)),
            pl.BlockSpec((BN, 2), lambda i: (i, 0)),
            pl.BlockSpec((BN, E), lambda i: (i, 0)),
        ],
        out_shape=[
            jax.ShapeDtypeStruct((N, 2), jnp.int32),
            jax.ShapeDtypeStruct((N, 2), jnp.float32),
            jax.ShapeDtypeStruct((N, E), jnp.float32),
        ],
        compiler_params=pltpu.CompilerParams(
            dimension_semantics=("arbitrary",),
        ),
    )(inputs, wt, b2)
    topk_indices, topk_gates, all_probabilities = out
    return (topk_indices, topk_gates, all_probabilities)


# speedup vs baseline: 1.6423x; 1.0156x over previous
import jax
import jax.numpy as jnp
from jax.experimental import pallas as pl
from jax.experimental.pallas import tpu as pltpu

N = 16384
D = 2048
E = 64
BN = 2048


def _router_kernel(x_ref, wt_ref, b_ref, idx_ref, gate_ref, prob_ref):
    x = x_ref[...]
    wt = wt_ref[...]
    b = b_ref[...]
    logits = jnp.dot(x, wt, preferred_element_type=jnp.float32) + b

    m = jnp.max(logits, axis=-1, keepdims=True)
    e = jnp.exp(logits - m)
    prob_ref[...] = e / jnp.sum(e, axis=-1, keepdims=True)

    i1 = jnp.argmax(logits, axis=-1)
    v1 = jnp.max(logits, axis=-1)
    lane = jax.lax.broadcasted_iota(jnp.int32, logits.shape, 1)
    masked = jnp.where(lane == i1[:, None], -jnp.inf, logits)
    i2 = jnp.argmax(masked, axis=-1)
    v2 = jnp.max(masked, axis=-1)

    idx_ref[...] = jnp.stack([i1, i2], axis=-1).astype(jnp.int32)

    g2 = 1.0 / (1.0 + jnp.exp(v1 - v2))
    g1 = 1.0 - g2
    gate_ref[...] = jnp.stack([g1, g2], axis=-1)


def kernel(inputs, W, b):
    wt = W.T
    b2 = b.reshape(1, E)
    grid = (N // BN,)
    out = pl.pallas_call(
        _router_kernel,
        grid=grid,
        in_specs=[
            pl.BlockSpec((BN, D), lambda i: (i, 0)),
            pl.BlockSpec((D, E), lambda i: (0, 0)),
            pl.BlockSpec((1, E), lambda i: (0, 0)),
        ],
        out_specs=[
            pl.BlockSpec((BN, 2), lambda i: (i, 0)),
            pl.BlockSpec((BN, 2), lambda i: (i, 0)),
            pl.BlockSpec((BN, E), lambda i: (i, 0)),
        ],
        out_shape=[
            jax.ShapeDtypeStruct((N, 2), jnp.int32),
            jax.ShapeDtypeStruct((N, 2), jnp.float32),
            jax.ShapeDtypeStruct((N, E), jnp.float32),
        ],
        compiler_params=pltpu.CompilerParams(
            dimension_semantics=("arbitrary",),
        ),
    )(inputs, wt, b2)
    topk_indices, topk_gates, all_probabilities = out
    return (topk_indices, topk_gates, all_probabilities)
